# Initial kernel scaffold; baseline (speedup 1.0000x reference)
import jax
import jax.numpy as jnp
from jax.experimental import pallas as pl


def kernel(x, neighbors, node_mask, graph_ids, W0, b0, Ws, bs, W_fc1, b_fc1, W_fc2, b_fc2):
    raise NotImplementedError("write your pallas kernel here")


# speedup vs baseline: 7.3647x; 7.3647x over previous
import functools

import jax
import jax.numpy as jnp
from jax import lax
from jax.experimental import pallas as pl
from jax.experimental.pallas import tpu as pltpu
from jax.experimental.pallas import tpu_sc as plsc

N = 100000
FL = 13
IN_C = 7
F = 16
L = 4
G = 1024

N_PAD = 100096
TOT = N_PAD * FL
NW = 32
G_CH = 23
G_K = 1768
SEG_NS = 16
SEG_CW = N_PAD // SEG_NS
SEG_GW = G // SEG_NS
RB = 4352


def _make_gather():
  mesh = plsc.VectorSubcoreMesh(core_axis_name="c", subcore_axis_name="s")

  @functools.partial(
      pl.kernel,
      mesh=mesh,
      out_type=jax.ShapeDtypeStruct((TOT, F), jnp.float32),
      scratch_types=[
          pltpu.VMEM((G_K,), jnp.int32),
          pltpu.VMEM((G_K, F), jnp.float32),
          pltpu.SemaphoreType.DMA,
      ],
      compiler_params=pltpu.CompilerParams(use_tc_tiling_on_sc=False),
  )
  def gather_k(table_hbm, idx_hbm, out_hbm, idx_v, rows_v, sem):
    wid = lax.axis_index("s") * 2 + lax.axis_index("c")
    base = wid * (G_CH * G_K)

    def body(c, carry):
      off = base + c * G_K
      pltpu.sync_copy(idx_hbm.at[pl.ds(off, G_K)], idx_v)
      pltpu.async_copy(table_hbm.at[idx_v], rows_v, sem).wait()
      pltpu.sync_copy(rows_v, out_hbm.at[pl.ds(off, G_K)])
      return carry

    lax.fori_loop(0, G_CH, body, 0)

  return gather_k


def _layer0_body(g_ref, w_ref, b_ref, h_ref, r_ref):
  h = jnp.dot(g_ref[...], w_ref[...], preferred_element_type=jnp.float32)
  h = h + b_ref[...]
  h_ref[...] = h
  r_ref[...] = jnp.maximum(h, 0.0)


def _layer_body(hp_ref, g_ref, w_ref, b_ref, h_ref, r_ref):
  h = jnp.dot(g_ref[...], w_ref[...], preferred_element_type=jnp.float32)
  h = hp_ref[...] + h + b_ref[...]
  h_ref[...] = h
  r_ref[...] = jnp.maximum(h, 0.0)


def _final_body(hp_ref, g_ref, w_ref, b_ref, w1_ref, b1_ref, w2_ref, b2_ref,
                mask_ref, y_ref):
  h = jnp.dot(g_ref[...], w_ref[...], preferred_element_type=jnp.float32)
  h = hp_ref[...] + h + b_ref[...]
  t = jnp.maximum(h, 0.0)
  t = jnp.dot(t, w1_ref[...], preferred_element_type=jnp.float32) + b1_ref[...]
  t = jnp.maximum(t, 0.0)
  y = jnp.dot(t, w2_ref[...], preferred_element_type=jnp.float32) + b2_ref[...]
  y_ref[...] = y * mask_ref[...]


def _row_spec(width):
  return pl.BlockSpec((RB, width), lambda i: (i, 0))


def _bcast_spec(shape):
  nd = len(shape)
  return pl.BlockSpec(shape, lambda i: (0,) * nd)


def _call_layer0(g, w, b):
  grid = (N_PAD // RB,)
  return pl.pallas_call(
      _layer0_body,
      grid=grid,
      in_specs=[_row_spec(FL * F), _bcast_spec(w.shape), _bcast_spec(b.shape)],
      out_specs=[_row_spec(F), _row_spec(F)],
      out_shape=[jax.ShapeDtypeStruct((N_PAD, F), jnp.float32)] * 2,
  )(g, w, b)


def _call_layer(hp, g, w, b):
  grid = (N_PAD // RB,)
  return pl.pallas_call(
      _layer_body,
      grid=grid,
      in_specs=[_row_spec(F), _row_spec(FL * F), _bcast_spec(w.shape),
                _bcast_spec(b.shape)],
      out_specs=[_row_spec(F), _row_spec(F)],
      out_shape=[jax.ShapeDtypeStruct((N_PAD, F), jnp.float32)] * 2,
  )(hp, g, w, b)


def _call_final(hp, g, w, b, w1, b1, w2, b2, mask):
  grid = (N_PAD // RB,)
  return pl.pallas_call(
      _final_body,
      grid=grid,
      in_specs=[_row_spec(F), _row_spec(FL * F), _bcast_spec(w.shape),
                _bcast_spec(b.shape), _bcast_spec(w1.shape),
                _bcast_spec(b1.shape), _bcast_spec(w2.shape),
                _bcast_spec(b2.shape), _row_spec(1)],
      out_specs=[_row_spec(1)],
      out_shape=[jax.ShapeDtypeStruct((N_PAD, 1), jnp.float32)],
  )(hp, g, w, b, w1, b1, w2, b2, mask)[0]


def _make_segment():
  mesh = plsc.VectorSubcoreMesh(
      core_axis_name="c", subcore_axis_name="s", num_cores=1)
  nvec = SEG_CW // 16

  @functools.partial(
      pl.kernel,
      mesh=mesh,
      out_type=jax.ShapeDtypeStruct((G,), jnp.float32),
      scratch_types=[
          pltpu.VMEM((SEG_CW,), jnp.float32),
          pltpu.VMEM((SEG_CW,), jnp.float32),
          pltpu.VMEM((SEG_CW,), jnp.int32),
          pltpu.VMEM((SEG_CW,), jnp.int32),
          pltpu.VMEM((4 * G,), jnp.float32),
          pltpu.VMEM((16,), jnp.float32),
          pltpu.VMEM((SEG_NS, 16), jnp.float32),
          pltpu.VMEM((4, SEG_NS, SEG_GW), jnp.float32),
          pltpu.VMEM((SEG_GW,), jnp.float32),
          pltpu.VMEM_SHARED((SEG_NS, 16), jnp.float32),
          pltpu.VMEM_SHARED((SEG_NS, 4 * G), jnp.float32),
      ],
      compiler_params=pltpu.CompilerParams(
          use_tc_tiling_on_sc=False, needs_layout_passes=False),
  )
  def seg_k(y_hbm, m_hbm, gid_hbm, gidn_hbm, out_hbm,
            y_v, m_v, gid_v, gidn_v, acc_v, tot_v, all_tot_v, comb_v, out_v,
            sh_tot, sh_acc):
    lanes = lax.broadcasted_iota(jnp.int32, (16,), 0)
    wid = lax.axis_index("s")
    base = wid * SEG_CW
    pltpu.sync_copy(y_hbm.at[pl.ds(base, SEG_CW)], y_v)
    pltpu.sync_copy(m_hbm.at[pl.ds(base, SEG_CW)], m_v)
    pltpu.sync_copy(gid_hbm.at[pl.ds(base, SEG_CW)], gid_v)
    pltpu.sync_copy(gidn_hbm.at[pl.ds(base, SEG_CW)], gidn_v)

    def tbody(i, carry):
      ty, tm = carry
      return (ty + jnp.sum(y_v[pl.ds(i * 16, 16)]),
              tm + jnp.sum(m_v[pl.ds(i * 16, 16)]))

    ty, tm = lax.fori_loop(0, nvec, tbody, (0.0, 0.0))
    tv = jnp.where(lanes == 0, jnp.full((16,), ty, jnp.float32),
                   jnp.where(lanes == 1, jnp.full((16,), tm, jnp.float32),
                             jnp.zeros((16,), jnp.float32)))
    tot_v[...] = tv
    pltpu.sync_copy(tot_v, sh_tot.at[wid])
    plsc.subcore_barrier()
    pltpu.sync_copy(sh_tot, all_tot_v)

    def cbody(w, carry):
      cy, cm = carry
      v = all_tot_v[w]
      take = (w < wid).astype(jnp.float32)
      e0 = jnp.sum(jnp.where(lanes == 0, v, 0.0))
      e1 = jnp.sum(jnp.where(lanes == 1, v, 0.0))
      return cy + take * e0, cm + take * e1

    cy0, cm0 = lax.fori_loop(0, SEG_NS, cbody, (0.0, 0.0))

    zv = jnp.zeros((16,), jnp.float32)

    def zbody(i, carry):
      acc_v[pl.ds(i * 16, 16)] = zv
      return carry

    lax.fori_loop(0, (4 * G) // 16, zbody, 0)

    def sbody(i, carry):
      cy, cm = carry
      yv = y_v[pl.ds(i * 16, 16)]
      mv = m_v[pl.ds(i * 16, 16)]
      gv = gid_v[pl.ds(i * 16, 16)]
      gnv = gidn_v[pl.ds(i * 16, 16)]
      ry = plsc.cumsum(yv) + cy
      rm = plsc.cumsum(mv) + cm
      bnd = gv != gnv
      mend = bnd & (gv < G)
      mstart = bnd & (gnv < G)
      plsc.store_scatter(acc_v, [gv], ry, mask=mend)
      plsc.store_scatter(acc_v, [gnv + G], ry, mask=mstart)
      plsc.store_scatter(acc_v, [gv + 2 * G], rm, mask=mend)
      plsc.store_scatter(acc_v, [gnv + 3 * G], rm, mask=mstart)
      return cy + jnp.sum(yv), cm + jnp.sum(mv)

    lax.fori_loop(0, nvec, sbody, (cy0, cm0))
    pltpu.sync_copy(acc_v, sh_acc.at[wid])
    plsc.subcore_barrier()

    for a in range(4):
      pltpu.sync_copy(sh_acc.at[:, pl.ds(a * G + wid * SEG_GW, SEG_GW)],
                      comb_v.at[a])

    def gbody(j, carry):
      def rbody(k, vals):
        ey, sy, em, sm = vals
        return (ey + comb_v[0, k, pl.ds(j * 16, 16)],
                sy + comb_v[1, k, pl.ds(j * 16, 16)],
                em + comb_v[2, k, pl.ds(j * 16, 16)],
                sm + comb_v[3, k, pl.ds(j * 16, 16)])

      z = jnp.zeros((16,), jnp.float32)
      ey, sy, em, sm = lax.fori_loop(0, SEG_NS, rbody, (z, z, z, z))
      num = ey - sy
      cnt = em - sm
      out_v[pl.ds(j * 16, 16)] = num / jnp.maximum(cnt, 1.0)
      return carry

    lax.fori_loop(0, SEG_GW // 16, gbody, 0)
    pltpu.sync_copy(out_v, out_hbm.at[pl.ds(wid * SEG_GW, SEG_GW)])

  return seg_k


def kernel(x, neighbors, node_mask, graph_ids, W0, b0, Ws, bs,
           W_fc1, b_fc1, W_fc2, b_fc2):
  f32 = jnp.float32
  xp = jnp.zeros((N_PAD, F), f32).at[:N, :IN_C].set(x)
  W0p = jnp.zeros((FL, F, F), f32).at[:, :IN_C, :].set(
      W0.reshape(FL, IN_C, F)).reshape(FL * F, F)
  idx = jnp.zeros((TOT,), jnp.int32).at[: N * FL].set(
      neighbors.astype(jnp.int32).reshape(-1))
  maskp = jnp.zeros((N_PAD, 1), f32).at[:N].set(node_mask.astype(f32))
  gid_ext = jnp.full((N_PAD + 8,), G, jnp.int32).at[:N].set(
      graph_ids.astype(jnp.int32))
  gid_a = gid_ext[:N_PAD]
  gid_b = gid_ext[1:N_PAD + 1]

  gather = _make_gather()
  b0r = b0.reshape(1, F)

  g = gather(xp, idx).reshape(N_PAD, FL * F)
  h, r = _call_layer0(g, W0p, b0r)
  for l in range(L - 1):
    g = gather(r, idx).reshape(N_PAD, FL * F)
    h, r = _call_layer(h, g, Ws[l], bs[l].reshape(1, F))
  g = gather(r, idx).reshape(N_PAD, FL * F)
  y = _call_final(h, g, Ws[L - 1], bs[L - 1].reshape(1, F),
                  W_fc1, b_fc1.reshape(1, 30), W_fc2, b_fc2.reshape(1, 1),
                  maskp)

  seg = _make_segment()
  out = seg(y.reshape(N_PAD), maskp.reshape(N_PAD), gid_a, gid_b)
  return out.reshape(G, 1)


# speedup vs baseline: 15.7116x; 2.1334x over previous
import functools

import jax
import jax.numpy as jnp
from jax import lax
from jax.experimental import pallas as pl
from jax.experimental.pallas import tpu as pltpu
from jax.experimental.pallas import tpu_sc as plsc

N = 100000
FL = 13
IN_C = 7
F = 16
L = 4
G = 1024

N_PAD = 100096
NP8 = N_PAD // 8
TOT = N_PAD * FL
NW = 32
G_CH = 23
G_K = 1768
SEG_NS = 16
SEG_CW = N_PAD // SEG_NS
SEG_GW = G // SEG_NS
RB8 = 544


def _make_gather():
  mesh = plsc.VectorSubcoreMesh(core_axis_name="c", subcore_axis_name="s")

  @functools.partial(
      pl.kernel,
      mesh=mesh,
      out_type=jax.ShapeDtypeStruct((TOT, F), jnp.float32),
      scratch_types=[
          pltpu.VMEM((G_K,), jnp.int32),
          pltpu.VMEM((G_K, F), jnp.float32),
          pltpu.SemaphoreType.DMA,
      ],
      compiler_params=pltpu.CompilerParams(use_tc_tiling_on_sc=False),
  )
  def gather_k(table_hbm, idx_hbm, out_hbm, idx_v, rows_v, sem):
    wid = lax.axis_index("s") * 2 + lax.axis_index("c")
    base = wid * (G_CH * G_K)

    def body(c, carry):
      off = base + c * G_K
      pltpu.sync_copy(idx_hbm.at[pl.ds(off, G_K)], idx_v)
      pltpu.async_copy(table_hbm.at[idx_v], rows_v, sem).wait()
      pltpu.sync_copy(rows_v, out_hbm.at[pl.ds(off, G_K)])
      return carry

    lax.fori_loop(0, G_CH, body, 0)

  return gather_k


def _accum_conv(g_ref, bw_ref):
  acc = jnp.dot(g_ref[0], bw_ref[0], preferred_element_type=jnp.float32)
  for j in range(1, FL):
    acc = acc + jnp.dot(g_ref[j], bw_ref[j],
                        preferred_element_type=jnp.float32)
  return acc


def _layer0_body(g_ref, bw_ref, b_ref, h_ref, r_ref):
  h = _accum_conv(g_ref, bw_ref) + b_ref[...]
  h_ref[...] = h
  r_ref[...] = jnp.maximum(h, 0.0)


def _layer_body(hp_ref, g_ref, bw_ref, b_ref, h_ref, r_ref):
  h = hp_ref[...] + _accum_conv(g_ref, bw_ref) + b_ref[...]
  h_ref[...] = h
  r_ref[...] = jnp.maximum(h, 0.0)


def _final_body(hp_ref, g_ref, bw_ref, b_ref, bw1_ref, b1_ref, bw2_ref,
                b2_ref, mask_ref, y_ref):
  h = hp_ref[...] + _accum_conv(g_ref, bw_ref) + b_ref[...]
  t = jnp.maximum(h, 0.0)
  t = jnp.dot(t, bw1_ref[...], preferred_element_type=jnp.float32)
  t = jnp.maximum(t + b1_ref[...], 0.0)
  y = jnp.dot(t, bw2_ref[...], preferred_element_type=jnp.float32)
  y_ref[...] = (y + b2_ref[...]) * mask_ref[...]


def _g_spec():
  return pl.BlockSpec((FL, RB8, 128), lambda i: (0, i, 0))


def _row_spec(width):
  return pl.BlockSpec((RB8, width), lambda i: (i, 0))


def _bcast_spec(shape):
  nd = len(shape)
  return pl.BlockSpec(shape, lambda i: (0,) * nd)


def _call_layer0(g128, bw, b):
  return pl.pallas_call(
      _layer0_body,
      grid=(NP8 // RB8,),
      in_specs=[_g_spec(), _bcast_spec(bw.shape), _bcast_spec(b.shape)],
      out_specs=[_row_spec(128), _row_spec(128)],
      out_shape=[jax.ShapeDtypeStruct((NP8, 128), jnp.float32)] * 2,
  )(g128, bw, b)


def _call_layer(hp, g128, bw, b):
  return pl.pallas_call(
      _layer_body,
      grid=(NP8 // RB8,),
      in_specs=[_row_spec(128), _g_spec(), _bcast_spec(bw.shape),
                _bcast_spec(b.shape)],
      out_specs=[_row_spec(128), _row_spec(128)],
      out_shape=[jax.ShapeDtypeStruct((NP8, 128), jnp.float32)] * 2,
  )(hp, g128, bw, b)


def _call_final(hp, g128, bw, b, bw1, b1, bw2, b2, mask8):
  return pl.pallas_call(
      _final_body,
      grid=(NP8 // RB8,),
      in_specs=[_row_spec(128), _g_spec(), _bcast_spec(bw.shape),
                _bcast_spec(b.shape), _bcast_spec(bw1.shape),
                _bcast_spec(b1.shape), _bcast_spec(bw2.shape),
                _bcast_spec(b2.shape), _row_spec(8)],
      out_specs=[_row_spec(8)],
      out_shape=[jax.ShapeDtypeStruct((NP8, 8), jnp.float32)],
  )(hp, g128, bw, b, bw1, b1, bw2, b2, mask8)[0]


def _make_segment():
  mesh = plsc.VectorSubcoreMesh(
      core_axis_name="c", subcore_axis_name="s", num_cores=1)
  nvec = SEG_CW // 16

  @functools.partial(
      pl.kernel,
      mesh=mesh,
      out_type=jax.ShapeDtypeStruct((G,), jnp.float32),
      scratch_types=[
          pltpu.VMEM((SEG_CW,), jnp.float32),
          pltpu.VMEM((SEG_CW,), jnp.float32),
          pltpu.VMEM((SEG_CW,), jnp.int32),
          pltpu.VMEM((SEG_CW,), jnp.int32),
          pltpu.VMEM((4 * G,), jnp.float32),
          pltpu.VMEM((16,), jnp.float32),
          pltpu.VMEM((SEG_NS, 16), jnp.float32),
          pltpu.VMEM((4, SEG_NS, SEG_GW), jnp.float32),
          pltpu.VMEM((SEG_GW,), jnp.float32),
          pltpu.VMEM_SHARED((SEG_NS, 16), jnp.float32),
          pltpu.VMEM_SHARED((SEG_NS, 4 * G), jnp.float32),
      ],
      compiler_params=pltpu.CompilerParams(
          use_tc_tiling_on_sc=False, needs_layout_passes=False),
  )
  def seg_k(y_hbm, m_hbm, gid_hbm, gidn_hbm, out_hbm,
            y_v, m_v, gid_v, gidn_v, acc_v, tot_v, all_tot_v, comb_v, out_v,
            sh_tot, sh_acc):
    lanes = lax.broadcasted_iota(jnp.int32, (16,), 0)
    wid = lax.axis_index("s")
    base = wid * SEG_CW
    pltpu.sync_copy(y_hbm.at[pl.ds(base, SEG_CW)], y_v)
    pltpu.sync_copy(m_hbm.at[pl.ds(base, SEG_CW)], m_v)
    pltpu.sync_copy(gid_hbm.at[pl.ds(base, SEG_CW)], gid_v)
    pltpu.sync_copy(gidn_hbm.at[pl.ds(base, SEG_CW)], gidn_v)

    def tbody(i, carry):
      ty, tm = carry
      return (ty + jnp.sum(y_v[pl.ds(i * 16, 16)]),
              tm + jnp.sum(m_v[pl.ds(i * 16, 16)]))

    ty, tm = lax.fori_loop(0, nvec, tbody, (0.0, 0.0))
    tv = jnp.where(lanes == 0, jnp.full((16,), ty, jnp.float32),
                   jnp.where(lanes == 1, jnp.full((16,), tm, jnp.float32),
                             jnp.zeros((16,), jnp.float32)))
    tot_v[...] = tv
    pltpu.sync_copy(tot_v, sh_tot.at[wid])
    plsc.subcore_barrier()
    pltpu.sync_copy(sh_tot, all_tot_v)

    def cbody(w, carry):
      cy, cm = carry
      v = all_tot_v[w]
      take = (w < wid).astype(jnp.float32)
      e0 = jnp.sum(jnp.where(lanes == 0, v, 0.0))
      e1 = jnp.sum(jnp.where(lanes == 1, v, 0.0))
      return cy + take * e0, cm + take * e1

    cy0, cm0 = lax.fori_loop(0, SEG_NS, cbody, (0.0, 0.0))

    zv = jnp.zeros((16,), jnp.float32)

    def zbody(i, carry):
      acc_v[pl.ds(i * 16, 16)] = zv
      return carry

    lax.fori_loop(0, (4 * G) // 16, zbody, 0)

    def sbody(i, carry):
      cy, cm = carry
      yv = y_v[pl.ds(i * 16, 16)]
      mv = m_v[pl.ds(i * 16, 16)]
      gv = gid_v[pl.ds(i * 16, 16)]
      gnv = gidn_v[pl.ds(i * 16, 16)]
      ry = plsc.cumsum(yv) + cy
      rm = plsc.cumsum(mv) + cm
      bnd = gv != gnv
      mend = bnd & (gv < G)
      mstart = bnd & (gnv < G)
      plsc.store_scatter(acc_v, [gv], ry, mask=mend)
      plsc.store_scatter(acc_v, [gnv + G], ry, mask=mstart)
      plsc.store_scatter(acc_v, [gv + 2 * G], rm, mask=mend)
      plsc.store_scatter(acc_v, [gnv + 3 * G], rm, mask=mstart)
      return cy + jnp.sum(yv), cm + jnp.sum(mv)

    lax.fori_loop(0, nvec, sbody, (cy0, cm0))
    pltpu.sync_copy(acc_v, sh_acc.at[wid])
    plsc.subcore_barrier()

    for a in range(4):
      pltpu.sync_copy(sh_acc.at[:, pl.ds(a * G + wid * SEG_GW, SEG_GW)],
                      comb_v.at[a])

    def gbody(j, carry):
      def rbody(k, vals):
        ey, sy, em, sm = vals
        return (ey + comb_v[0, k, pl.ds(j * 16, 16)],
                sy + comb_v[1, k, pl.ds(j * 16, 16)],
                em + comb_v[2, k, pl.ds(j * 16, 16)],
                sm + comb_v[3, k, pl.ds(j * 16, 16)])

      z = jnp.zeros((16,), jnp.float32)
      ey, sy, em, sm = lax.fori_loop(0, SEG_NS, rbody, (z, z, z, z))
      num = ey - sy
      cnt = em - sm
      out_v[pl.ds(j * 16, 16)] = num / jnp.maximum(cnt, 1.0)
      return carry

    lax.fori_loop(0, SEG_GW // 16, gbody, 0)
    pltpu.sync_copy(out_v, out_hbm.at[pl.ds(wid * SEG_GW, SEG_GW)])

  return seg_k


def _block_diag(w):
  eye8 = jnp.eye(8, dtype=jnp.float32)
  return jnp.einsum("ab,jcf->jacbf", eye8, w).reshape(FL, 128, 128)


def kernel(x, neighbors, node_mask, graph_ids, W0, b0, Ws, bs,
           W_fc1, b_fc1, W_fc2, b_fc2):
  f32 = jnp.float32
  eye8 = jnp.eye(8, dtype=f32)
  xp = jnp.zeros((N_PAD, F), f32).at[:N, :IN_C].set(x)
  idxT = jnp.zeros((FL, N_PAD), jnp.int32).at[:, :N].set(
      neighbors.astype(jnp.int32).T).reshape(TOT)
  maskp = jnp.zeros((N_PAD,), f32).at[:N].set(node_mask[:, 0].astype(f32))
  mask8 = maskp.reshape(NP8, 8)
  gid_ext = jnp.full((N_PAD + 8,), G, jnp.int32).at[:N].set(
      graph_ids.astype(jnp.int32))
  gid_a = gid_ext[:N_PAD]
  gid_b = gid_ext[1:N_PAD + 1]

  W0p = jnp.zeros((FL, F, F), f32).at[:, :IN_C, :].set(
      W0.reshape(FL, IN_C, F))
  bw0 = _block_diag(W0p)
  bwl = [_block_diag(Ws[l].reshape(FL, F, F)) for l in range(L)]
  bb = [jnp.tile(b, 8).reshape(1, 128) for b in (b0, *bs)]
  bw1 = jnp.einsum("ab,cf->acbf", eye8, W_fc1).reshape(128, 240)
  b1p = jnp.tile(b_fc1, 8).reshape(1, 240)
  bw2 = jnp.einsum("ab,c->acb", eye8, W_fc2[:, 0]).reshape(240, 8)
  b2p = jnp.tile(b_fc2, 8).reshape(1, 8)

  gather = _make_gather()

  g = gather(xp, idxT).reshape(FL, NP8, 128)
  h, r = _call_layer0(g, bw0, bb[0])
  for l in range(L - 1):
    g = gather(r.reshape(N_PAD, F), idxT).reshape(FL, NP8, 128)
    h, r = _call_layer(h, g, bwl[l], bb[l + 1])
  g = gather(r.reshape(N_PAD, F), idxT).reshape(FL, NP8, 128)
  y = _call_final(h, g, bwl[L - 1], bb[L], bw1, b1p, bw2, b2p, mask8)

  seg = _make_segment()
  out = seg(y.reshape(N_PAD), maskp, gid_a, gid_b)
  return out.reshape(G, 1)


# speedup vs baseline: 19.0279x; 1.2111x over previous
import functools

import jax
import jax.numpy as jnp
from jax import lax
from jax.experimental import pallas as pl
from jax.experimental.pallas import tpu as pltpu
from jax.experimental.pallas import tpu_sc as plsc

N = 100000
FL = 13
IN_C = 7
F = 16
L = 4
G = 1024

N_PAD = 100096
NP8 = N_PAD // 8
TOT = N_PAD * FL
NW = 32
G_CH = 13
G_K = 3128
SEG_NS = 16
SEG_CW = N_PAD // SEG_NS
SEG_GW = G // SEG_NS
RB8 = 544


def _make_gather():
  mesh = plsc.VectorSubcoreMesh(core_axis_name="c", subcore_axis_name="s")

  @functools.partial(
      pl.kernel,
      mesh=mesh,
      out_type=jax.ShapeDtypeStruct((TOT, F), jnp.float32),
      scratch_types=[
          pltpu.VMEM((G_K,), jnp.int32),
          pltpu.VMEM((G_K,), jnp.int32),
          pltpu.VMEM((G_K, F), jnp.float32),
          pltpu.VMEM((G_K, F), jnp.float32),
          pltpu.SemaphoreType.DMA,
          pltpu.SemaphoreType.DMA,
          pltpu.SemaphoreType.DMA,
          pltpu.SemaphoreType.DMA,
          pltpu.SemaphoreType.DMA,
          pltpu.SemaphoreType.DMA,
      ],
      compiler_params=pltpu.CompilerParams(use_tc_tiling_on_sc=False),
  )
  def gather_k(table_hbm, idx_hbm, out_hbm,
               idx0, idx1, rows0, rows1, is0, is1, gs0, gs1, ws0, ws1):
    tbl = table_hbm
    wid = lax.axis_index("s") * 2 + lax.axis_index("c")
    base = wid * (G_CH * G_K)
    idx_v = (idx0, idx1)
    rows_v = (rows0, rows1)
    isem = (is0, is1)
    gsem = (gs0, gs1)
    wsem = (ws0, ws1)
    gd = [None, None]
    wd = [None, None]
    idd = [None, None]

    idd[0] = pltpu.async_copy(idx_hbm.at[pl.ds(base, G_K)], idx_v[0], isem[0])
    idd[1] = pltpu.async_copy(idx_hbm.at[pl.ds(base + G_K, G_K)], idx_v[1],
                              isem[1])
    for c in range(G_CH):
      b = c % 2
      o = 1 - b
      if c >= 1:
        gd[o].wait()
        wd[o] = pltpu.async_copy(
            rows_v[o], out_hbm.at[pl.ds(base + (c - 1) * G_K, G_K)], wsem[o])
        if c + 1 < G_CH:
          idd[o] = pltpu.async_copy(
              idx_hbm.at[pl.ds(base + (c + 1) * G_K, G_K)], idx_v[o], isem[o])
      if c >= 2:
        wd[b].wait()
      idd[b].wait()
      gd[b] = pltpu.async_copy(tbl.at[idx_v[b]], rows_v[b], gsem[b])
    bl = (G_CH - 1) % 2
    gd[bl].wait()
    pltpu.sync_copy(rows_v[bl], out_hbm.at[pl.ds(base + (G_CH - 1) * G_K, G_K)])

  return gather_k


def _accum_conv(g_ref, bw_ref):
  acc = jnp.dot(g_ref[0], bw_ref[0], preferred_element_type=jnp.float32)
  for j in range(1, FL):
    acc = acc + jnp.dot(g_ref[j], bw_ref[j],
                        preferred_element_type=jnp.float32)
  return acc


def _layer0_body(g_ref, bw_ref, b_ref, h_ref, r_ref):
  h = _accum_conv(g_ref, bw_ref) + b_ref[...]
  h_ref[...] = h
  r_ref[...] = jnp.maximum(h, 0.0)


def _layer_body(hp_ref, g_ref, bw_ref, b_ref, h_ref, r_ref):
  h = hp_ref[...] + _accum_conv(g_ref, bw_ref) + b_ref[...]
  h_ref[...] = h
  r_ref[...] = jnp.maximum(h, 0.0)


def _final_body(hp_ref, g_ref, bw_ref, b_ref, bw1_ref, b1_ref, bw2_ref,
                b2_ref, mask_ref, y_ref):
  h = hp_ref[...] + _accum_conv(g_ref, bw_ref) + b_ref[...]
  t = jnp.maximum(h, 0.0)
  t = jnp.dot(t, bw1_ref[...], preferred_element_type=jnp.float32)
  t = jnp.maximum(t + b1_ref[...], 0.0)
  y = jnp.dot(t, bw2_ref[...], preferred_element_type=jnp.float32)
  y_ref[...] = (y + b2_ref[...]) * mask_ref[...]


def _g_spec():
  return pl.BlockSpec((FL, RB8, 128), lambda i: (0, i, 0))


def _row_spec(width):
  return pl.BlockSpec((RB8, width), lambda i: (i, 0))


def _bcast_spec(shape):
  nd = len(shape)
  return pl.BlockSpec(shape, lambda i: (0,) * nd)


def _call_layer0(g128, bw, b):
  return pl.pallas_call(
      _layer0_body,
      grid=(NP8 // RB8,),
      in_specs=[_g_spec(), _bcast_spec(bw.shape), _bcast_spec(b.shape)],
      out_specs=[_row_spec(128), _row_spec(128)],
      out_shape=[jax.ShapeDtypeStruct((NP8, 128), jnp.float32)] * 2,
  )(g128, bw, b)


def _call_layer(hp, g128, bw, b):
  return pl.pallas_call(
      _layer_body,
      grid=(NP8 // RB8,),
      in_specs=[_row_spec(128), _g_spec(), _bcast_spec(bw.shape),
                _bcast_spec(b.shape)],
      out_specs=[_row_spec(128), _row_spec(128)],
      out_shape=[jax.ShapeDtypeStruct((NP8, 128), jnp.float32)] * 2,
  )(hp, g128, bw, b)


def _call_final(hp, g128, bw, b, bw1, b1, bw2, b2, mask8):
  return pl.pallas_call(
      _final_body,
      grid=(NP8 // RB8,),
      in_specs=[_row_spec(128), _g_spec(), _bcast_spec(bw.shape),
                _bcast_spec(b.shape), _bcast_spec(bw1.shape),
                _bcast_spec(b1.shape), _bcast_spec(bw2.shape),
                _bcast_spec(b2.shape), _row_spec(8)],
      out_specs=[_row_spec(8)],
      out_shape=[jax.ShapeDtypeStruct((NP8, 8), jnp.float32)],
  )(hp, g128, bw, b, bw1, b1, bw2, b2, mask8)[0]


def _make_segment():
  mesh = plsc.VectorSubcoreMesh(
      core_axis_name="c", subcore_axis_name="s", num_cores=1)
  nvec = SEG_CW // 16

  @functools.partial(
      pl.kernel,
      mesh=mesh,
      out_type=jax.ShapeDtypeStruct((G,), jnp.float32),
      scratch_types=[
          pltpu.VMEM((SEG_CW,), jnp.float32),
          pltpu.VMEM((SEG_CW,), jnp.float32),
          pltpu.VMEM((SEG_CW,), jnp.int32),
          pltpu.VMEM((SEG_CW,), jnp.int32),
          pltpu.VMEM((4 * G,), jnp.float32),
          pltpu.VMEM((16,), jnp.float32),
          pltpu.VMEM((SEG_NS, 16), jnp.float32),
          pltpu.VMEM((4, SEG_NS, SEG_GW), jnp.float32),
          pltpu.VMEM((SEG_GW,), jnp.float32),
          pltpu.VMEM_SHARED((SEG_NS, 16), jnp.float32),
          pltpu.VMEM_SHARED((SEG_NS, 4 * G), jnp.float32),
      ],
      compiler_params=pltpu.CompilerParams(
          use_tc_tiling_on_sc=False, needs_layout_passes=False),
  )
  def seg_k(y_hbm, m_hbm, gid_hbm, gidn_hbm, out_hbm,
            y_v, m_v, gid_v, gidn_v, acc_v, tot_v, all_tot_v, comb_v, out_v,
            sh_tot, sh_acc):
    lanes = lax.broadcasted_iota(jnp.int32, (16,), 0)
    wid = lax.axis_index("s")
    base = wid * SEG_CW
    pltpu.sync_copy(y_hbm.at[pl.ds(base, SEG_CW)], y_v)
    pltpu.sync_copy(m_hbm.at[pl.ds(base, SEG_CW)], m_v)
    pltpu.sync_copy(gid_hbm.at[pl.ds(base, SEG_CW)], gid_v)
    pltpu.sync_copy(gidn_hbm.at[pl.ds(base, SEG_CW)], gidn_v)

    def tbody(i, carry):
      ty, tm = carry
      return (ty + jnp.sum(y_v[pl.ds(i * 16, 16)]),
              tm + jnp.sum(m_v[pl.ds(i * 16, 16)]))

    ty, tm = lax.fori_loop(0, nvec, tbody, (0.0, 0.0))
    tv = jnp.where(lanes == 0, jnp.full((16,), ty, jnp.float32),
                   jnp.where(lanes == 1, jnp.full((16,), tm, jnp.float32),
                             jnp.zeros((16,), jnp.float32)))
    tot_v[...] = tv
    pltpu.sync_copy(tot_v, sh_tot.at[wid])
    plsc.subcore_barrier()
    pltpu.sync_copy(sh_tot, all_tot_v)

    def cbody(w, carry):
      cy, cm = carry
      v = all_tot_v[w]
      take = (w < wid).astype(jnp.float32)
      e0 = jnp.sum(jnp.where(lanes == 0, v, 0.0))
      e1 = jnp.sum(jnp.where(lanes == 1, v, 0.0))
      return cy + take * e0, cm + take * e1

    cy0, cm0 = lax.fori_loop(0, SEG_NS, cbody, (0.0, 0.0))

    zv = jnp.zeros((16,), jnp.float32)

    def zbody(i, carry):
      acc_v[pl.ds(i * 16, 16)] = zv
      return carry

    lax.fori_loop(0, (4 * G) // 16, zbody, 0)

    def sbody(i, carry):
      cy, cm = carry
      yv = y_v[pl.ds(i * 16, 16)]
      mv = m_v[pl.ds(i * 16, 16)]
      gv = gid_v[pl.ds(i * 16, 16)]
      gnv = gidn_v[pl.ds(i * 16, 16)]
      ry = plsc.cumsum(yv) + cy
      rm = plsc.cumsum(mv) + cm
      bnd = gv != gnv
      mend = bnd & (gv < G)
      mstart = bnd & (gnv < G)
      plsc.store_scatter(acc_v, [gv], ry, mask=mend)
      plsc.store_scatter(acc_v, [gnv + G], ry, mask=mstart)
      plsc.store_scatter(acc_v, [gv + 2 * G], rm, mask=mend)
      plsc.store_scatter(acc_v, [gnv + 3 * G], rm, mask=mstart)
      return cy + jnp.sum(yv), cm + jnp.sum(mv)

    lax.fori_loop(0, nvec, sbody, (cy0, cm0))
    pltpu.sync_copy(acc_v, sh_acc.at[wid])
    plsc.subcore_barrier()

    for a in range(4):
      pltpu.sync_copy(sh_acc.at[:, pl.ds(a * G + wid * SEG_GW, SEG_GW)],
                      comb_v.at[a])

    def gbody(j, carry):
      def rbody(k, vals):
        ey, sy, em, sm = vals
        return (ey + comb_v[0, k, pl.ds(j * 16, 16)],
                sy + comb_v[1, k, pl.ds(j * 16, 16)],
                em + comb_v[2, k, pl.ds(j * 16, 16)],
                sm + comb_v[3, k, pl.ds(j * 16, 16)])

      z = jnp.zeros((16,), jnp.float32)
      ey, sy, em, sm = lax.fori_loop(0, SEG_NS, rbody, (z, z, z, z))
      num = ey - sy
      cnt = em - sm
      out_v[pl.ds(j * 16, 16)] = num / jnp.maximum(cnt, 1.0)
      return carry

    lax.fori_loop(0, SEG_GW // 16, gbody, 0)
    pltpu.sync_copy(out_v, out_hbm.at[pl.ds(wid * SEG_GW, SEG_GW)])

  return seg_k


def _block_diag(w):
  eye8 = jnp.eye(8, dtype=jnp.float32)
  return jnp.einsum("ab,jcf->jacbf", eye8, w).reshape(FL, 128, 128)


def kernel(x, neighbors, node_mask, graph_ids, W0, b0, Ws, bs,
           W_fc1, b_fc1, W_fc2, b_fc2):
  f32 = jnp.float32
  eye8 = jnp.eye(8, dtype=f32)
  xp8 = jnp.pad(x, ((0, N_PAD - N), (0, F - IN_C))).reshape(NP8, 128)
  idxT = jnp.zeros((FL, N_PAD), jnp.int32).at[:, :N].set(
      neighbors.astype(jnp.int32).T).reshape(TOT)
  maskp = jnp.zeros((N_PAD,), f32).at[:N].set(node_mask[:, 0].astype(f32))
  mask8 = maskp.reshape(NP8, 8)
  gid_ext = jnp.full((N_PAD + 8,), G, jnp.int32).at[:N].set(
      graph_ids.astype(jnp.int32))
  gid_a = gid_ext[:N_PAD]
  gid_b = gid_ext[1:N_PAD + 1]

  W0p = jnp.zeros((FL, F, F), f32).at[:, :IN_C, :].set(
      W0.reshape(FL, IN_C, F))
  bw0 = _block_diag(W0p)
  bwl = [_block_diag(Ws[l].reshape(FL, F, F)) for l in range(L)]
  bb = [jnp.tile(b, 8).reshape(1, 128) for b in (b0, *bs)]
  bw1 = jnp.einsum("ab,cf->acbf", eye8, W_fc1).reshape(128, 240)
  b1p = jnp.tile(b_fc1, 8).reshape(1, 240)
  bw2 = jnp.einsum("ab,c->acb", eye8, W_fc2[:, 0]).reshape(240, 8)
  b2p = jnp.tile(b_fc2, 8).reshape(1, 8)

  gather = _make_gather()

  g = gather(xp8.reshape(N_PAD, F), idxT).reshape(FL, NP8, 128)
  h, r = _call_layer0(g, bw0, bb[0])
  for l in range(L - 1):
    g = gather(r.reshape(N_PAD, F), idxT).reshape(FL, NP8, 128)
    h, r = _call_layer(h, g, bwl[l], bb[l + 1])
  g = gather(r.reshape(N_PAD, F), idxT).reshape(FL, NP8, 128)
  y = _call_final(h, g, bwl[L - 1], bb[L], bw1, b1p, bw2, b2p, mask8)

  seg = _make_segment()
  out = seg(y.reshape(N_PAD), maskp, gid_a, gid_b)
  return out.reshape(G, 1)
